# Initial kernel scaffold; baseline (speedup 1.0000x reference)
import functools

import jax
import jax.numpy as jnp
from jax import lax
from jax.experimental import pallas as pl
from jax.experimental.pallas import tpu as pltpu
from jax.experimental.pallas import tpu_sc as plsc

_N = 10000
_E = 320000
_D = 128
_ED = 16
_G = 16

_NC = 2
_NS = 16
_EPT = _E // (_NC * _NS)
_K = 80
_NCH = _EPT // _K
_RPT = _N // _NS
_ZR = 25
_NZ = _RPT // _ZR


def _sc_agg_body(x_hbm, ea_hbm, src_hbm, dst_hbm, out_hbm,
                 src_v, dst_v, xr_v, ear_v, zb_v, acc_sh, sem):
    cid = lax.axis_index("c")
    sid = lax.axis_index("s")

    @pl.loop(0, _ZR)
    def _zfill(i):
        for j in range(_D // 16):
            zb_v[i, pl.ds(j * 16, 16)] = jnp.zeros((16,), jnp.float32)

    row0 = sid * _RPT

    @pl.loop(0, _NZ)
    def _zcopy(k):
        pltpu.sync_copy(zb_v, acc_sh.at[pl.ds(row0 + k * _ZR, _ZR)])

    plsc.subcore_barrier()

    ebase = (cid * _NS + sid) * _EPT

    @pl.loop(0, _NCH)
    def _chunk(ch):
        b = ebase + ch * _K
        pltpu.sync_copy(src_hbm.at[pl.ds(b, _K)], src_v)
        pltpu.sync_copy(dst_hbm.at[pl.ds(b, _K)], dst_v)
        pltpu.async_copy(x_hbm.at[src_v], xr_v, sem).wait()
        pltpu.sync_copy(ea_hbm.at[pl.ds(b, _K)], ear_v)

        @pl.loop(0, _K)
        def _row(i):
            for j in range(_D // 16):
                s = pl.ds(j * 16, 16)
                xr_v[i, s] = jnp.maximum(xr_v[i, s] + ear_v[i, s], 0.0)

        pltpu.sync_copy(xr_v, acc_sh.at[dst_v], add=True)

    plsc.subcore_barrier()

    @pl.loop(0, _NZ)
    def _wb(k):
        r = row0 + k * _ZR
        pltpu.sync_copy(acc_sh.at[pl.ds(r, _ZR)], zb_v)
        pltpu.sync_copy(zb_v, out_hbm.at[cid, pl.ds(r, _ZR)])


_sc_agg = functools.partial(
    pl.kernel,
    out_type=jax.ShapeDtypeStruct((_NC, _N, _D), jnp.float32),
    mesh=plsc.VectorSubcoreMesh(core_axis_name="c", subcore_axis_name="s"),
    scratch_types=[
        pltpu.VMEM((_K,), jnp.int32),
        pltpu.VMEM((_K,), jnp.int32),
        pltpu.VMEM((_K, _D), jnp.float32),
        pltpu.VMEM((_K, _D), jnp.float32),
        pltpu.VMEM((_ZR, _D), jnp.float32),
        pltpu.VMEM_SHARED((_N, _D), jnp.float32),
        pltpu.SemaphoreType.DMA,
    ],
)(_sc_agg_body)


def _silu(x):
    return x * jax.nn.sigmoid(x)


_BE = 2000


def _edge_mlp_body(ea_ref, We1_ref, be1_ref, We2_ref, be2_ref, o1_ref, o2_ref):
    a = ea_ref[...]
    h1 = jnp.dot(a, We1_ref[...], preferred_element_type=jnp.float32) + be1_ref[...]
    o1_ref[...] = _silu(h1)
    h2 = jnp.dot(a, We2_ref[...], preferred_element_type=jnp.float32) + be2_ref[...]
    o2_ref[...] = _silu(h2)


def _edge_mlp(edge_attr, We1, be1, We2, be2):
    return pl.pallas_call(
        _edge_mlp_body,
        grid=(_E // _BE,),
        in_specs=[
            pl.BlockSpec((_BE, _ED), lambda i: (i, 0)),
            pl.BlockSpec((_ED, _D), lambda i: (0, 0)),
            pl.BlockSpec((1, _D), lambda i: (0, 0)),
            pl.BlockSpec((_ED, _D), lambda i: (0, 0)),
            pl.BlockSpec((1, _D), lambda i: (0, 0)),
        ],
        out_specs=[
            pl.BlockSpec((_BE, _D), lambda i: (i, 0)),
            pl.BlockSpec((_BE, _D), lambda i: (i, 0)),
        ],
        out_shape=[
            jax.ShapeDtypeStruct((_E, _D), jnp.float32),
            jax.ShapeDtypeStruct((_E, _D), jnp.float32),
        ],
    )(edge_attr, We1, be1.reshape(1, _D), We2, be2.reshape(1, _D))


def _node1_body(aggp_ref, x_ref, n2g_ref, Wa_ref, ba_ref, Wb_ref, bb_ref,
                gamma_ref, beta_ref, scale_ref, o_ref):
    s = scale_ref[0, 0]
    upd = aggp_ref[0] + aggp_ref[1] + s * x_ref[...]
    h = _silu(jnp.dot(upd, Wa_ref[...], preferred_element_type=jnp.float32) + ba_ref[...])
    h = _silu(jnp.dot(h, Wb_ref[...], preferred_element_type=jnp.float32) + bb_ref[...])
    gids = lax.broadcasted_iota(jnp.int32, (_N, _G), 1)
    onehot = (n2g_ref[...] == gids).astype(jnp.float32)
    ones = jnp.ones((_N, 1), jnp.float32)
    cnt = lax.dot_general(onehot, ones, (((0,), (0,)), ((), ())),
                          preferred_element_type=jnp.float32)
    denom = jnp.maximum(cnt * float(_D), 1.0)
    sums = lax.dot_general(onehot, h, (((0,), (0,)), ((), ())),
                           preferred_element_type=jnp.float32)
    mean = jnp.sum(sums, axis=1, keepdims=True) / denom
    mean_n = jnp.dot(onehot, mean, preferred_element_type=jnp.float32)
    xc = h - mean_n
    sq = lax.dot_general(onehot, xc * xc, (((0,), (0,)), ((), ())),
                         preferred_element_type=jnp.float32)
    var = jnp.sum(sq, axis=1, keepdims=True) / denom
    rs = lax.rsqrt(var + 1e-5)
    rs_n = jnp.dot(onehot, rs, preferred_element_type=jnp.float32)
    out = (xc * rs_n) * gamma_ref[...] + beta_ref[...]
    o_ref[...] = jnp.maximum(out, 0.0)


def _node1(aggp, x, n2g, Wa, ba, Wb, bb, gamma, beta, scale):
    return pl.pallas_call(
        _node1_body,
        out_shape=jax.ShapeDtypeStruct((_N, _D), jnp.float32),
    )(aggp, x, n2g.reshape(_N, 1), Wa, ba.reshape(1, _D), Wb, bb.reshape(1, _D),
      gamma.reshape(1, _D), beta.reshape(1, _D), scale)


def _node2_body(aggp_ref, h1_ref, x_ref, Wa_ref, ba_ref, Wb_ref, bb_ref,
                scale_ref, o_ref):
    s = scale_ref[0, 0]
    upd = aggp_ref[0] + aggp_ref[1] + s * h1_ref[...]
    h = _silu(jnp.dot(upd, Wa_ref[...], preferred_element_type=jnp.float32) + ba_ref[...])
    h = _silu(jnp.dot(h, Wb_ref[...], preferred_element_type=jnp.float32) + bb_ref[...])
    o_ref[...] = jnp.maximum((h + x_ref[...]) * 0.5, 0.0)


def _node2(aggp, h1, x, Wa, ba, Wb, bb, scale):
    return pl.pallas_call(
        _node2_body,
        out_shape=jax.ShapeDtypeStruct((_N, _D), jnp.float32),
    )(aggp, h1, x, Wa, ba.reshape(1, _D), Wb, bb.reshape(1, _D), scale)


def kernel(x, edge_index, edge_attr, node2graph, We1, be1, eps1, W1a, b1a,
           W1b, b1b, gamma1, beta1, We2, be2, eps2, W2a, b2a, W2b, b2b):
    src = edge_index[0]
    dst = edge_index[1]
    scale1 = jnp.reshape(1.0 + eps1, (1, 1)).astype(jnp.float32)
    scale2 = jnp.reshape(1.0 + eps2, (1, 1)).astype(jnp.float32)

    ea1, ea2 = _edge_mlp(edge_attr, We1, be1, We2, be2)
    aggp1 = _sc_agg(x, ea1, src, dst)
    h1 = _node1(aggp1, x, node2graph, W1a, b1a, W1b, b1b, gamma1, beta1, scale1)
    aggp2 = _sc_agg(h1, ea2, src, dst)
    return _node2(aggp2, h1, x, W2a, b2a, W2b, b2b, scale2)


# speedup vs baseline: 2.2644x; 2.2644x over previous
import functools

import jax
import jax.numpy as jnp
from jax import lax
from jax.experimental import pallas as pl
from jax.experimental.pallas import tpu as pltpu
from jax.experimental.pallas import tpu_sc as plsc

_N = 10000
_E = 320000
_D = 128
_ED = 16
_G = 16

_NC = 2
_NS = 16
_EPT = _E // (_NC * _NS)
_K = 80
_NCH = _EPT // _K
_RPT = 624
_ZR = 16


def _sc_agg_body(x_hbm, ea_hbm, src_hbm, dst_hbm, out_hbm,
                 src_v, dst_v, xr_v, ear_v, zb_v, acc_sh, sem):
    cid = lax.axis_index("c")
    sid = lax.axis_index("s")

    @pl.loop(0, _ZR)
    def _zfill(i):
        for j in range(_D // 16):
            zb_v[i, pl.ds(j * 16, 16)] = jnp.zeros((16,), jnp.float32)

    row0 = sid * _RPT
    nz = jnp.where(sid == _NS - 1, (_N - (_NS - 1) * _RPT) // _ZR, _RPT // _ZR)

    @pl.loop(0, nz)
    def _zcopy(k):
        pltpu.sync_copy(zb_v, acc_sh.at[pl.ds(row0 + k * _ZR, _ZR)])

    plsc.subcore_barrier()

    ebase = (cid * _NS + sid) * _EPT

    @pl.loop(0, _NCH)
    def _chunk(ch):
        b = ebase + ch * _K
        pltpu.sync_copy(src_hbm.at[pl.ds(b, _K)], src_v)
        pltpu.sync_copy(dst_hbm.at[pl.ds(b, _K)], dst_v)
        pltpu.async_copy(x_hbm.at[src_v], xr_v, sem).wait()
        pltpu.sync_copy(ea_hbm.at[pl.ds(b, _K)], ear_v)

        @pl.loop(0, _K)
        def _row(i):
            for j in range(_D // 16):
                s = pl.ds(j * 16, 16)
                xr_v[i, s] = jnp.maximum(xr_v[i, s] + ear_v[i, s], 0.0)

        pltpu.sync_copy(xr_v, acc_sh.at[dst_v], add=True)

    plsc.subcore_barrier()

    @pl.loop(0, nz)
    def _wb(k):
        r = row0 + k * _ZR
        pltpu.sync_copy(acc_sh.at[pl.ds(r, _ZR)], zb_v)
        pltpu.sync_copy(zb_v, out_hbm.at[cid, pl.ds(r, _ZR)])


@functools.cache
def _make_sc_agg():
    return pl.kernel(
        _sc_agg_body,
        out_type=jax.ShapeDtypeStruct((_NC, _N, _D), jnp.float32),
        mesh=plsc.VectorSubcoreMesh(core_axis_name="c", subcore_axis_name="s",
                                    num_cores=_NC, num_subcores=_NS),
        scratch_types=[
            pltpu.VMEM((_K,), jnp.int32),
            pltpu.VMEM((_K,), jnp.int32),
            pltpu.VMEM((_K, _D), jnp.float32),
            pltpu.VMEM((_K, _D), jnp.float32),
            pltpu.VMEM((_ZR, _D), jnp.float32),
            pltpu.VMEM_SHARED((_N, _D), jnp.float32),
            pltpu.SemaphoreType.DMA,
        ],
    )


def _sc_agg(x, ea, src, dst):
    return _make_sc_agg()(x, ea, src, dst)


def _silu(x):
    return x * jax.nn.sigmoid(x)


_BE = 2000


def _edge_mlp_body(ea_ref, We1_ref, be1_ref, We2_ref, be2_ref, o1_ref, o2_ref):
    a = ea_ref[...]
    h1 = jnp.dot(a, We1_ref[...], preferred_element_type=jnp.float32, precision=lax.Precision.HIGHEST) + be1_ref[...]
    o1_ref[...] = _silu(h1)
    h2 = jnp.dot(a, We2_ref[...], preferred_element_type=jnp.float32, precision=lax.Precision.HIGHEST) + be2_ref[...]
    o2_ref[...] = _silu(h2)


def _edge_mlp(edge_attr, We1, be1, We2, be2):
    return pl.pallas_call(
        _edge_mlp_body,
        grid=(_E // _BE,),
        in_specs=[
            pl.BlockSpec((_BE, _ED), lambda i: (i, 0)),
            pl.BlockSpec((_ED, _D), lambda i: (0, 0)),
            pl.BlockSpec((1, _D), lambda i: (0, 0)),
            pl.BlockSpec((_ED, _D), lambda i: (0, 0)),
            pl.BlockSpec((1, _D), lambda i: (0, 0)),
        ],
        out_specs=[
            pl.BlockSpec((_BE, _D), lambda i: (i, 0)),
            pl.BlockSpec((_BE, _D), lambda i: (i, 0)),
        ],
        out_shape=[
            jax.ShapeDtypeStruct((_E, _D), jnp.float32),
            jax.ShapeDtypeStruct((_E, _D), jnp.float32),
        ],
    )(edge_attr, We1, be1.reshape(1, _D), We2, be2.reshape(1, _D))


def _node1_body(aggp_ref, x_ref, n2g_ref, Wa_ref, ba_ref, Wb_ref, bb_ref,
                gamma_ref, beta_ref, scale_ref, o_ref):
    s = scale_ref[0, 0]
    upd = aggp_ref[0] + aggp_ref[1] + s * x_ref[...]
    h = _silu(jnp.dot(upd, Wa_ref[...], preferred_element_type=jnp.float32, precision=lax.Precision.HIGHEST) + ba_ref[...])
    h = _silu(jnp.dot(h, Wb_ref[...], preferred_element_type=jnp.float32, precision=lax.Precision.HIGHEST) + bb_ref[...])
    gids = lax.broadcasted_iota(jnp.int32, (_N, _G), 1)
    onehot = (n2g_ref[...] == gids).astype(jnp.float32)
    ones = jnp.ones((_N, 1), jnp.float32)
    cnt = lax.dot_general(onehot, ones, (((0,), (0,)), ((), ())),
                          preferred_element_type=jnp.float32, precision=lax.Precision.HIGHEST)
    denom = jnp.maximum(cnt * float(_D), 1.0)
    sums = lax.dot_general(onehot, h, (((0,), (0,)), ((), ())),
                           preferred_element_type=jnp.float32, precision=lax.Precision.HIGHEST)
    mean = jnp.sum(sums, axis=1, keepdims=True) / denom
    mean_n = jnp.dot(onehot, mean, preferred_element_type=jnp.float32, precision=lax.Precision.HIGHEST)
    xc = h - mean_n
    sq = lax.dot_general(onehot, xc * xc, (((0,), (0,)), ((), ())),
                         preferred_element_type=jnp.float32, precision=lax.Precision.HIGHEST)
    var = jnp.sum(sq, axis=1, keepdims=True) / denom
    rs = lax.rsqrt(var + 1e-5)
    rs_n = jnp.dot(onehot, rs, preferred_element_type=jnp.float32, precision=lax.Precision.HIGHEST)
    out = (xc * rs_n) * gamma_ref[...] + beta_ref[...]
    o_ref[...] = jnp.maximum(out, 0.0)


def _node1(aggp, x, n2g, Wa, ba, Wb, bb, gamma, beta, scale):
    return pl.pallas_call(
        _node1_body,
        out_shape=jax.ShapeDtypeStruct((_N, _D), jnp.float32),
    )(aggp, x, n2g.reshape(_N, 1), Wa, ba.reshape(1, _D), Wb, bb.reshape(1, _D),
      gamma.reshape(1, _D), beta.reshape(1, _D), scale)


def _node2_body(aggp_ref, h1_ref, x_ref, Wa_ref, ba_ref, Wb_ref, bb_ref,
                scale_ref, o_ref):
    s = scale_ref[0, 0]
    upd = aggp_ref[0] + aggp_ref[1] + s * h1_ref[...]
    h = _silu(jnp.dot(upd, Wa_ref[...], preferred_element_type=jnp.float32, precision=lax.Precision.HIGHEST) + ba_ref[...])
    h = _silu(jnp.dot(h, Wb_ref[...], preferred_element_type=jnp.float32, precision=lax.Precision.HIGHEST) + bb_ref[...])
    o_ref[...] = jnp.maximum((h + x_ref[...]) * 0.5, 0.0)


def _node2(aggp, h1, x, Wa, ba, Wb, bb, scale):
    return pl.pallas_call(
        _node2_body,
        out_shape=jax.ShapeDtypeStruct((_N, _D), jnp.float32),
    )(aggp, h1, x, Wa, ba.reshape(1, _D), Wb, bb.reshape(1, _D), scale)


def kernel(x, edge_index, edge_attr, node2graph, We1, be1, eps1, W1a, b1a,
           W1b, b1b, gamma1, beta1, We2, be2, eps2, W2a, b2a, W2b, b2b):
    src = edge_index[0]
    dst = edge_index[1]
    scale1 = jnp.reshape(1.0 + eps1, (1, 1)).astype(jnp.float32)
    scale2 = jnp.reshape(1.0 + eps2, (1, 1)).astype(jnp.float32)

    ea1, ea2 = _edge_mlp(edge_attr, We1, be1, We2, be2)
    aggp1 = _sc_agg(x, ea1, src, dst)
    h1 = _node1(aggp1, x, node2graph, W1a, b1a, W1b, b1b, gamma1, beta1, scale1)
    aggp2 = _sc_agg(h1, ea2, src, dst)
    return _node2(aggp2, h1, x, W2a, b2a, W2b, b2b, scale2)


# speedup vs baseline: 3.5430x; 1.5647x over previous
import functools

import jax
import jax.numpy as jnp
from jax import lax
from jax.experimental import pallas as pl
from jax.experimental.pallas import tpu as pltpu
from jax.experimental.pallas import tpu_sc as plsc

_N = 10000
_E = 320000
_D = 128
_ED = 16
_G = 16

_NC = 2
_NS = 16
_EPT = _E // (_NC * _NS)
_K = 80
_NCH = _EPT // _K
_RPT = 624
_ZR = 16


def _sc_agg_body(x_hbm, ea_hbm, src_hbm, dst2_hbm, out_hbm,
                 srcall_v, dst2_v, data0, data1, zb_v, acc_sh,
                 sem_g0, sem_g1, sem_e0, sem_e1):
    cid = lax.axis_index("c")
    sid = lax.axis_index("s")
    ebase = (cid * _NS + sid) * _EPT
    tid = cid * _NS + sid

    pltpu.sync_copy(src_hbm.at[pl.ds(ebase, _EPT)], srcall_v)
    pltpu.sync_copy(dst2_hbm.at[tid], dst2_v)

    @pl.loop(0, _ZR)
    def _zfill(i):
        for j in range(_D // 16):
            zb_v[i, pl.ds(j * 16, 16)] = jnp.zeros((16,), jnp.float32)

    row0 = sid * _RPT
    nz = jnp.where(sid == _NS - 1, (_N - (_NS - 1) * _RPT) // _ZR, _RPT // _ZR)

    @pl.loop(0, nz)
    def _zcopy(k):
        pltpu.sync_copy(zb_v, acc_sh.at[pl.ds(row0 + k * _ZR, _ZR)])

    plsc.subcore_barrier()

    bufs = ((data0, sem_g0, sem_e0), (data1, sem_g1, sem_e1))

    def ea_slice(c):
        return ea_hbm.at[pl.ds(ebase + c * _K, _K)]

    def gx_src(c):
        return x_hbm.at[srcall_v.at[pl.ds(c * _K, _K)]]

    def relu_scatter(data, c):
        @pl.loop(0, _K)
        def _row(i):
            for j in range(_D // 16):
                s = pl.ds(j * 16, 16)
                data[i, s] = jnp.maximum(data[i, s], 0.0)

        pltpu.sync_copy(data, acc_sh.at[dst2_v.at[c]], add=True)

    @pl.loop(0, _NCH // 2)
    def _pair(p):
        c0 = p * 2
        e0 = pltpu.async_copy(ea_slice(c0), data0, sem_e0)
        e1 = pltpu.async_copy(ea_slice(c0 + 1), data1, sem_e1)
        e0.wait()
        g0 = pltpu.async_copy(gx_src(c0), data0, sem_g0, add=True)
        e1.wait()
        g1 = pltpu.async_copy(gx_src(c0 + 1), data1, sem_g1, add=True)
        g0.wait()
        relu_scatter(data0, c0)
        g1.wait()
        relu_scatter(data1, c0 + 1)

    c_last = _NCH - 1
    pltpu.sync_copy(ea_slice(c_last), data0)
    pltpu.async_copy(gx_src(c_last), data0, sem_g0, add=True).wait()
    relu_scatter(data0, c_last)

    plsc.subcore_barrier()

    @pl.loop(0, nz)
    def _wb(k):
        r = row0 + k * _ZR
        pltpu.sync_copy(acc_sh.at[pl.ds(r, _ZR)], zb_v)
        pltpu.sync_copy(zb_v, out_hbm.at[cid, pl.ds(r, _ZR)])


@functools.cache
def _make_sc_agg():
    return pl.kernel(
        _sc_agg_body,
        out_type=jax.ShapeDtypeStruct((_NC, _N, _D), jnp.float32),
        mesh=plsc.VectorSubcoreMesh(core_axis_name="c", subcore_axis_name="s",
                                    num_cores=_NC, num_subcores=_NS),
        scratch_types=[
            pltpu.VMEM((_EPT,), jnp.int32),
            pltpu.VMEM((_NCH, _K), jnp.int32),
            pltpu.VMEM((_K, _D), jnp.float32),
            pltpu.VMEM((_K, _D), jnp.float32),
            pltpu.VMEM((_ZR, _D), jnp.float32),
            pltpu.VMEM_SHARED((_N, _D), jnp.float32),
            pltpu.SemaphoreType.DMA,
            pltpu.SemaphoreType.DMA,
            pltpu.SemaphoreType.DMA,
            pltpu.SemaphoreType.DMA,
        ],
    )


def _sc_agg(x, ea, src, dst2):
    return _make_sc_agg()(x, ea, src, dst2)


def _silu(x):
    return x * jax.nn.sigmoid(x)


_BE = 2000


def _edge_mlp_body(ea_ref, We_ref, be_ref, o_ref):
    a = ea_ref[...]
    h = jnp.dot(a, We_ref[...], preferred_element_type=jnp.float32, precision=lax.Precision.HIGHEST) + be_ref[...]
    o_ref[...] = _silu(h)


def _edge_mlp(edge_attr, We, be):
    return pl.pallas_call(
        _edge_mlp_body,
        grid=(_E // _BE,),
        in_specs=[
            pl.BlockSpec((_BE, _ED), lambda i: (i, 0)),
            pl.BlockSpec((_ED, _D), lambda i: (0, 0)),
            pl.BlockSpec((1, _D), lambda i: (0, 0)),
        ],
        out_specs=pl.BlockSpec((_BE, _D), lambda i: (i, 0)),
        out_shape=jax.ShapeDtypeStruct((_E, _D), jnp.float32),
    )(edge_attr, We, be.reshape(1, _D))


def _node1_body(aggp_ref, x_ref, n2g_ref, Wa_ref, ba_ref, Wb_ref, bb_ref,
                gamma_ref, beta_ref, scale_ref, o_ref):
    s = scale_ref[0, 0]
    upd = aggp_ref[0] + aggp_ref[1] + s * x_ref[...]
    h = _silu(jnp.dot(upd, Wa_ref[...], preferred_element_type=jnp.float32, precision=lax.Precision.HIGHEST) + ba_ref[...])
    h = _silu(jnp.dot(h, Wb_ref[...], preferred_element_type=jnp.float32, precision=lax.Precision.HIGHEST) + bb_ref[...])
    gids = lax.broadcasted_iota(jnp.int32, (_N, _G), 1)
    onehot = (n2g_ref[...] == gids).astype(jnp.float32)
    ones = jnp.ones((_N, 1), jnp.float32)
    cnt = lax.dot_general(onehot, ones, (((0,), (0,)), ((), ())),
                          preferred_element_type=jnp.float32, precision=lax.Precision.HIGHEST)
    denom = jnp.maximum(cnt * float(_D), 1.0)
    sums = lax.dot_general(onehot, h, (((0,), (0,)), ((), ())),
                           preferred_element_type=jnp.float32, precision=lax.Precision.HIGHEST)
    mean = jnp.sum(sums, axis=1, keepdims=True) / denom
    mean_n = jnp.dot(onehot, mean, preferred_element_type=jnp.float32, precision=lax.Precision.HIGHEST)
    xc = h - mean_n
    sq = lax.dot_general(onehot, xc * xc, (((0,), (0,)), ((), ())),
                         preferred_element_type=jnp.float32, precision=lax.Precision.HIGHEST)
    var = jnp.sum(sq, axis=1, keepdims=True) / denom
    rs = lax.rsqrt(var + 1e-5)
    rs_n = jnp.dot(onehot, rs, preferred_element_type=jnp.float32, precision=lax.Precision.HIGHEST)
    out = (xc * rs_n) * gamma_ref[...] + beta_ref[...]
    o_ref[...] = jnp.maximum(out, 0.0)


def _node1(aggp, x, n2g, Wa, ba, Wb, bb, gamma, beta, scale):
    return pl.pallas_call(
        _node1_body,
        out_shape=jax.ShapeDtypeStruct((_N, _D), jnp.float32),
    )(aggp, x, n2g.reshape(_N, 1), Wa, ba.reshape(1, _D), Wb, bb.reshape(1, _D),
      gamma.reshape(1, _D), beta.reshape(1, _D), scale)


def _node2_body(aggp_ref, h1_ref, x_ref, Wa_ref, ba_ref, Wb_ref, bb_ref,
                scale_ref, o_ref):
    s = scale_ref[0, 0]
    upd = aggp_ref[0] + aggp_ref[1] + s * h1_ref[...]
    h = _silu(jnp.dot(upd, Wa_ref[...], preferred_element_type=jnp.float32, precision=lax.Precision.HIGHEST) + ba_ref[...])
    h = _silu(jnp.dot(h, Wb_ref[...], preferred_element_type=jnp.float32, precision=lax.Precision.HIGHEST) + bb_ref[...])
    o_ref[...] = jnp.maximum((h + x_ref[...]) * 0.5, 0.0)


def _node2(aggp, h1, x, Wa, ba, Wb, bb, scale):
    return pl.pallas_call(
        _node2_body,
        out_shape=jax.ShapeDtypeStruct((_N, _D), jnp.float32),
    )(aggp, h1, x, Wa, ba.reshape(1, _D), Wb, bb.reshape(1, _D), scale)


def kernel(x, edge_index, edge_attr, node2graph, We1, be1, eps1, W1a, b1a,
           W1b, b1b, gamma1, beta1, We2, be2, eps2, W2a, b2a, W2b, b2b):
    src = edge_index[0]
    dst2 = edge_index[1].reshape(_NC * _NS, _NCH, _K)
    scale1 = jnp.reshape(1.0 + eps1, (1, 1)).astype(jnp.float32)
    scale2 = jnp.reshape(1.0 + eps2, (1, 1)).astype(jnp.float32)

    ea1 = _edge_mlp(edge_attr, We1, be1)
    aggp1 = _sc_agg(x, ea1, src, dst2)
    ea2 = _edge_mlp(edge_attr, We2, be2)
    h1 = _node1(aggp1, x, node2graph, W1a, b1a, W1b, b1b, gamma1, beta1, scale1)
    aggp2 = _sc_agg(h1, ea2, src, dst2)
    return _node2(aggp2, h1, x, W2a, b2a, W2b, b2b, scale2)


# speedup vs baseline: 3.7858x; 1.0685x over previous
import functools

import jax
import jax.numpy as jnp
from jax import lax
from jax.experimental import pallas as pl
from jax.experimental.pallas import tpu as pltpu
from jax.experimental.pallas import tpu_sc as plsc

_N = 10000
_E = 320000
_D = 128
_ED = 16
_G = 16

_NC = 2
_NS = 16
_EPT = _E // (_NC * _NS)
_K = 80
_NCH = _EPT // _K
_RPT = 624
_ZR = 16


def _sc_agg_body(x_hbm, ea_hbm, src_hbm, dst2_hbm, out_hbm,
                 srcall_v, dst2_v, data0, data1, zb_v, acc_sh,
                 sem_g0, sem_g1, sem_e0, sem_e1):
    cid = lax.axis_index("c")
    sid = lax.axis_index("s")
    ebase = (cid * _NS + sid) * _EPT
    tid = cid * _NS + sid

    pltpu.sync_copy(src_hbm.at[pl.ds(ebase, _EPT)], srcall_v)
    pltpu.sync_copy(dst2_hbm.at[tid], dst2_v)

    @pl.loop(0, _ZR)
    def _zfill(i):
        for j in range(_D // 16):
            zb_v[i, pl.ds(j * 16, 16)] = jnp.zeros((16,), jnp.float32)

    row0 = sid * _RPT
    nz = jnp.where(sid == _NS - 1, (_N - (_NS - 1) * _RPT) // _ZR, _RPT // _ZR)

    @pl.loop(0, nz)
    def _zcopy(k):
        pltpu.sync_copy(zb_v, acc_sh.at[pl.ds(row0 + k * _ZR, _ZR)])

    plsc.subcore_barrier()

    bufs = ((data0, sem_g0, sem_e0), (data1, sem_g1, sem_e1))

    def ea_slice(c):
        return ea_hbm.at[pl.ds(ebase + c * _K, _K)]

    def gx_src(c):
        return x_hbm.at[srcall_v.at[pl.ds(c * _K, _K)]]

    def relu_scatter(data, c):
        @pl.loop(0, _K)
        def _row(i):
            for j in range(_D // 16):
                s = pl.ds(j * 16, 16)
                data[i, s] = jnp.maximum(data[i, s], 0.0)

        pltpu.sync_copy(data, acc_sh.at[dst2_v.at[c]], add=True)

    pltpu.sync_copy(ea_slice(0), data0)
    pltpu.async_copy(gx_src(0), data0, sem_g0, add=True)
    pltpu.async_copy(ea_slice(1), data1, sem_e1)

    @pl.loop(0, (_NCH - 1) // 2)
    def _pair(p):
        for b in range(2):
            c = p * 2 + b
            data, sem_g, _ = bufs[b]
            dataN, sem_gN, sem_eN = bufs[1 - b]
            pltpu.make_async_copy(gx_src(c), data, sem_g).wait()
            pltpu.make_async_copy(ea_slice(c + 1), dataN, sem_eN).wait()
            pltpu.async_copy(gx_src(c + 1), dataN, sem_gN, add=True)
            relu_scatter(data, c)

            @pl.when(c + 2 < _NCH)
            def _():
                pltpu.async_copy(ea_slice(c + 2), data, bufs[b][2])

    c_last = _NCH - 1
    pltpu.make_async_copy(gx_src(c_last), data0, sem_g0).wait()
    relu_scatter(data0, c_last)

    plsc.subcore_barrier()

    @pl.loop(0, nz)
    def _wb(k):
        r = row0 + k * _ZR
        pltpu.sync_copy(acc_sh.at[pl.ds(r, _ZR)], zb_v)
        pltpu.sync_copy(zb_v, out_hbm.at[cid, pl.ds(r, _ZR)])


@functools.cache
def _make_sc_agg():
    return pl.kernel(
        _sc_agg_body,
        out_type=jax.ShapeDtypeStruct((_NC, _N, _D), jnp.float32),
        mesh=plsc.VectorSubcoreMesh(core_axis_name="c", subcore_axis_name="s",
                                    num_cores=_NC, num_subcores=_NS),
        scratch_types=[
            pltpu.VMEM((_EPT,), jnp.int32),
            pltpu.VMEM((_NCH, _K), jnp.int32),
            pltpu.VMEM((_K, _D), jnp.float32),
            pltpu.VMEM((_K, _D), jnp.float32),
            pltpu.VMEM((_ZR, _D), jnp.float32),
            pltpu.VMEM_SHARED((_N, _D), jnp.float32),
            pltpu.SemaphoreType.DMA,
            pltpu.SemaphoreType.DMA,
            pltpu.SemaphoreType.DMA,
            pltpu.SemaphoreType.DMA,
        ],
    )


def _sc_agg(x, ea, src, dst2):
    return _make_sc_agg()(x, ea, src, dst2)


def _silu(x):
    return x * jax.nn.sigmoid(x)


_BE = 2000


def _edge_mlp_body(ea_ref, We_ref, be_ref, o_ref):
    a = ea_ref[...]
    h = jnp.dot(a, We_ref[...], preferred_element_type=jnp.float32, precision=lax.Precision.HIGHEST) + be_ref[...]
    o_ref[...] = _silu(h)


def _edge_mlp(edge_attr, We, be):
    return pl.pallas_call(
        _edge_mlp_body,
        grid=(_E // _BE,),
        in_specs=[
            pl.BlockSpec((_BE, _ED), lambda i: (i, 0)),
            pl.BlockSpec((_ED, _D), lambda i: (0, 0)),
            pl.BlockSpec((1, _D), lambda i: (0, 0)),
        ],
        out_specs=pl.BlockSpec((_BE, _D), lambda i: (i, 0)),
        out_shape=jax.ShapeDtypeStruct((_E, _D), jnp.float32),
    )(edge_attr, We, be.reshape(1, _D))


def _node1_body(aggp_ref, x_ref, n2g_ref, Wa_ref, ba_ref, Wb_ref, bb_ref,
                gamma_ref, beta_ref, scale_ref, o_ref):
    s = scale_ref[0, 0]
    upd = aggp_ref[0] + aggp_ref[1] + s * x_ref[...]
    h = _silu(jnp.dot(upd, Wa_ref[...], preferred_element_type=jnp.float32, precision=lax.Precision.HIGHEST) + ba_ref[...])
    h = _silu(jnp.dot(h, Wb_ref[...], preferred_element_type=jnp.float32, precision=lax.Precision.HIGHEST) + bb_ref[...])
    gids = lax.broadcasted_iota(jnp.int32, (_N, _G), 1)
    onehot = (n2g_ref[...] == gids).astype(jnp.float32)
    ones = jnp.ones((_N, 1), jnp.float32)
    cnt = lax.dot_general(onehot, ones, (((0,), (0,)), ((), ())),
                          preferred_element_type=jnp.float32, precision=lax.Precision.HIGHEST)
    denom = jnp.maximum(cnt * float(_D), 1.0)
    sums = lax.dot_general(onehot, h, (((0,), (0,)), ((), ())),
                           preferred_element_type=jnp.float32, precision=lax.Precision.HIGHEST)
    mean = jnp.sum(sums, axis=1, keepdims=True) / denom
    mean_n = jnp.dot(onehot, mean, preferred_element_type=jnp.float32, precision=lax.Precision.HIGHEST)
    xc = h - mean_n
    sq = lax.dot_general(onehot, xc * xc, (((0,), (0,)), ((), ())),
                         preferred_element_type=jnp.float32, precision=lax.Precision.HIGHEST)
    var = jnp.sum(sq, axis=1, keepdims=True) / denom
    rs = lax.rsqrt(var + 1e-5)
    rs_n = jnp.dot(onehot, rs, preferred_element_type=jnp.float32, precision=lax.Precision.HIGHEST)
    out = (xc * rs_n) * gamma_ref[...] + beta_ref[...]
    o_ref[...] = jnp.maximum(out, 0.0)


def _node1(aggp, x, n2g, Wa, ba, Wb, bb, gamma, beta, scale):
    return pl.pallas_call(
        _node1_body,
        out_shape=jax.ShapeDtypeStruct((_N, _D), jnp.float32),
    )(aggp, x, n2g.reshape(_N, 1), Wa, ba.reshape(1, _D), Wb, bb.reshape(1, _D),
      gamma.reshape(1, _D), beta.reshape(1, _D), scale)


def _node2_body(aggp_ref, h1_ref, x_ref, Wa_ref, ba_ref, Wb_ref, bb_ref,
                scale_ref, o_ref):
    s = scale_ref[0, 0]
    upd = aggp_ref[0] + aggp_ref[1] + s * h1_ref[...]
    h = _silu(jnp.dot(upd, Wa_ref[...], preferred_element_type=jnp.float32, precision=lax.Precision.HIGHEST) + ba_ref[...])
    h = _silu(jnp.dot(h, Wb_ref[...], preferred_element_type=jnp.float32, precision=lax.Precision.HIGHEST) + bb_ref[...])
    o_ref[...] = jnp.maximum((h + x_ref[...]) * 0.5, 0.0)


def _node2(aggp, h1, x, Wa, ba, Wb, bb, scale):
    return pl.pallas_call(
        _node2_body,
        out_shape=jax.ShapeDtypeStruct((_N, _D), jnp.float32),
    )(aggp, h1, x, Wa, ba.reshape(1, _D), Wb, bb.reshape(1, _D), scale)


def kernel(x, edge_index, edge_attr, node2graph, We1, be1, eps1, W1a, b1a,
           W1b, b1b, gamma1, beta1, We2, be2, eps2, W2a, b2a, W2b, b2b):
    src = edge_index[0]
    dst2 = edge_index[1].reshape(_NC * _NS, _NCH, _K)
    scale1 = jnp.reshape(1.0 + eps1, (1, 1)).astype(jnp.float32)
    scale2 = jnp.reshape(1.0 + eps2, (1, 1)).astype(jnp.float32)

    ea1 = _edge_mlp(edge_attr, We1, be1)
    aggp1 = _sc_agg(x, ea1, src, dst2)
    ea2 = _edge_mlp(edge_attr, We2, be2)
    h1 = _node1(aggp1, x, node2graph, W1a, b1a, W1b, b1b, gamma1, beta1, scale1)
    aggp2 = _sc_agg(h1, ea2, src, dst2)
    return _node2(aggp2, h1, x, W2a, b2a, W2b, b2b, scale2)


# speedup vs baseline: 3.7935x; 1.0020x over previous
import functools

import jax
import jax.numpy as jnp
from jax import lax
from jax.experimental import pallas as pl
from jax.experimental.pallas import tpu as pltpu
from jax.experimental.pallas import tpu_sc as plsc

_N = 10000
_E = 320000
_D = 128
_ED = 16
_G = 16

_NC = 2
_NS = 16
_EPT = _E // (_NC * _NS)
_K = 80
_NCH = _EPT // _K
_RPT = 624
_ZR = 16


def _sc_agg_body(x_hbm, ea_hbm, src_hbm, dst2_hbm, out_hbm,
                 srcall_v, dst2_v, data0, data1, zb_v, acc_sh,
                 sem_g0, sem_g1, sem_e0, sem_e1):
    cid = lax.axis_index("c")
    sid = lax.axis_index("s")
    ebase = (cid * _NS + sid) * _EPT
    tid = cid * _NS + sid

    pltpu.sync_copy(src_hbm.at[pl.ds(ebase, _EPT)], srcall_v)
    pltpu.sync_copy(dst2_hbm.at[tid], dst2_v)

    @pl.loop(0, _ZR)
    def _zfill(i):
        for j in range(_D // 16):
            zb_v[i, pl.ds(j * 16, 16)] = jnp.zeros((16,), jnp.float32)

    row0 = sid * _RPT
    nz = jnp.where(sid == _NS - 1, (_N - (_NS - 1) * _RPT) // _ZR, _RPT // _ZR)

    @pl.loop(0, nz)
    def _zcopy(k):
        pltpu.sync_copy(zb_v, acc_sh.at[pl.ds(row0 + k * _ZR, _ZR)])

    plsc.subcore_barrier()

    bufs = ((data0, sem_g0, sem_e0), (data1, sem_g1, sem_e1))

    def ea_slice(c):
        return ea_hbm.at[pl.ds(ebase + c * _K, _K)]

    def gx_src(c):
        return x_hbm.at[srcall_v.at[pl.ds(c * _K, _K)]]

    def relu_scatter(data, c):
        @pl.loop(0, _K, unroll=4)
        def _row(i):
            for j in range(_D // 16):
                s = pl.ds(j * 16, 16)
                data[i, s] = jnp.maximum(data[i, s], 0.0)

        pltpu.sync_copy(data, acc_sh.at[dst2_v.at[c]], add=True)

    pltpu.sync_copy(ea_slice(0), data0)
    pltpu.async_copy(gx_src(0), data0, sem_g0, add=True)
    pltpu.async_copy(ea_slice(1), data1, sem_e1)

    @pl.loop(0, (_NCH - 1) // 2)
    def _pair(p):
        for b in range(2):
            c = p * 2 + b
            data, sem_g, _ = bufs[b]
            dataN, sem_gN, sem_eN = bufs[1 - b]
            pltpu.make_async_copy(gx_src(c), data, sem_g).wait()
            pltpu.make_async_copy(ea_slice(c + 1), dataN, sem_eN).wait()
            pltpu.async_copy(gx_src(c + 1), dataN, sem_gN, add=True)
            relu_scatter(data, c)

            @pl.when(c + 2 < _NCH)
            def _():
                pltpu.async_copy(ea_slice(c + 2), data, bufs[b][2])

    c_last = _NCH - 1
    pltpu.make_async_copy(gx_src(c_last), data0, sem_g0).wait()
    relu_scatter(data0, c_last)

    plsc.subcore_barrier()

    @pl.loop(0, nz)
    def _wb(k):
        r = row0 + k * _ZR
        pltpu.sync_copy(acc_sh.at[pl.ds(r, _ZR)], zb_v)
        pltpu.sync_copy(zb_v, out_hbm.at[cid, pl.ds(r, _ZR)])


@functools.cache
def _make_sc_agg():
    return pl.kernel(
        _sc_agg_body,
        out_type=jax.ShapeDtypeStruct((_NC, _N, _D), jnp.float32),
        mesh=plsc.VectorSubcoreMesh(core_axis_name="c", subcore_axis_name="s",
                                    num_cores=_NC, num_subcores=_NS),
        scratch_types=[
            pltpu.VMEM((_EPT,), jnp.int32),
            pltpu.VMEM((_NCH, _K), jnp.int32),
            pltpu.VMEM((_K, _D), jnp.float32),
            pltpu.VMEM((_K, _D), jnp.float32),
            pltpu.VMEM((_ZR, _D), jnp.float32),
            pltpu.VMEM_SHARED((_N, _D), jnp.float32),
        ] + [pltpu.SemaphoreType.DMA] * 4,
    )


def _sc_agg(x, ea, src, dst2):
    return _make_sc_agg()(x, ea, src, dst2)


def _silu(x):
    return x * jax.nn.sigmoid(x)


_BE = 2000


def _edge_mlp_body(ea_ref, We_ref, be_ref, o_ref):
    a = ea_ref[...]
    h = jnp.dot(a, We_ref[...], preferred_element_type=jnp.float32, precision=lax.Precision.HIGHEST) + be_ref[...]
    o_ref[...] = _silu(h)


def _edge_mlp(edge_attr, We, be):
    return pl.pallas_call(
        _edge_mlp_body,
        grid=(_E // _BE,),
        in_specs=[
            pl.BlockSpec((_BE, _ED), lambda i: (i, 0)),
            pl.BlockSpec((_ED, _D), lambda i: (0, 0)),
            pl.BlockSpec((1, _D), lambda i: (0, 0)),
        ],
        out_specs=pl.BlockSpec((_BE, _D), lambda i: (i, 0)),
        out_shape=jax.ShapeDtypeStruct((_E, _D), jnp.float32),
    )(edge_attr, We, be.reshape(1, _D))


def _node1_body(aggp_ref, x_ref, n2g_ref, Wa_ref, ba_ref, Wb_ref, bb_ref,
                gamma_ref, beta_ref, scale_ref, o_ref):
    s = scale_ref[0, 0]
    upd = aggp_ref[0] + aggp_ref[1] + s * x_ref[...]
    h = _silu(jnp.dot(upd, Wa_ref[...], preferred_element_type=jnp.float32, precision=lax.Precision.HIGHEST) + ba_ref[...])
    h = _silu(jnp.dot(h, Wb_ref[...], preferred_element_type=jnp.float32, precision=lax.Precision.HIGHEST) + bb_ref[...])
    gids = lax.broadcasted_iota(jnp.int32, (_N, _G), 1)
    onehot = (n2g_ref[...] == gids).astype(jnp.float32)
    ones = jnp.ones((_N, 1), jnp.float32)
    cnt = lax.dot_general(onehot, ones, (((0,), (0,)), ((), ())),
                          preferred_element_type=jnp.float32, precision=lax.Precision.HIGHEST)
    denom = jnp.maximum(cnt * float(_D), 1.0)
    sums = lax.dot_general(onehot, h, (((0,), (0,)), ((), ())),
                           preferred_element_type=jnp.float32, precision=lax.Precision.HIGHEST)
    mean = jnp.sum(sums, axis=1, keepdims=True) / denom
    mean_n = jnp.dot(onehot, mean, preferred_element_type=jnp.float32, precision=lax.Precision.HIGHEST)
    xc = h - mean_n
    sq = lax.dot_general(onehot, xc * xc, (((0,), (0,)), ((), ())),
                         preferred_element_type=jnp.float32, precision=lax.Precision.HIGHEST)
    var = jnp.sum(sq, axis=1, keepdims=True) / denom
    rs = lax.rsqrt(var + 1e-5)
    rs_n = jnp.dot(onehot, rs, preferred_element_type=jnp.float32, precision=lax.Precision.HIGHEST)
    out = (xc * rs_n) * gamma_ref[...] + beta_ref[...]
    o_ref[...] = jnp.maximum(out, 0.0)


def _node1(aggp, x, n2g, Wa, ba, Wb, bb, gamma, beta, scale):
    return pl.pallas_call(
        _node1_body,
        out_shape=jax.ShapeDtypeStruct((_N, _D), jnp.float32),
    )(aggp, x, n2g.reshape(_N, 1), Wa, ba.reshape(1, _D), Wb, bb.reshape(1, _D),
      gamma.reshape(1, _D), beta.reshape(1, _D), scale)


def _node2_body(aggp_ref, h1_ref, x_ref, Wa_ref, ba_ref, Wb_ref, bb_ref,
                scale_ref, o_ref):
    s = scale_ref[0, 0]
    upd = aggp_ref[0] + aggp_ref[1] + s * h1_ref[...]
    h = _silu(jnp.dot(upd, Wa_ref[...], preferred_element_type=jnp.float32, precision=lax.Precision.HIGHEST) + ba_ref[...])
    h = _silu(jnp.dot(h, Wb_ref[...], preferred_element_type=jnp.float32, precision=lax.Precision.HIGHEST) + bb_ref[...])
    o_ref[...] = jnp.maximum((h + x_ref[...]) * 0.5, 0.0)


def _node2(aggp, h1, x, Wa, ba, Wb, bb, scale):
    return pl.pallas_call(
        _node2_body,
        out_shape=jax.ShapeDtypeStruct((_N, _D), jnp.float32),
    )(aggp, h1, x, Wa, ba.reshape(1, _D), Wb, bb.reshape(1, _D), scale)


def kernel(x, edge_index, edge_attr, node2graph, We1, be1, eps1, W1a, b1a,
           W1b, b1b, gamma1, beta1, We2, be2, eps2, W2a, b2a, W2b, b2b):
    src = edge_index[0]
    dst2 = edge_index[1].reshape(_NC * _NS, _NCH, _K)
    scale1 = jnp.reshape(1.0 + eps1, (1, 1)).astype(jnp.float32)
    scale2 = jnp.reshape(1.0 + eps2, (1, 1)).astype(jnp.float32)

    ea1 = _edge_mlp(edge_attr, We1, be1)
    aggp1 = _sc_agg(x, ea1, src, dst2)
    ea2 = _edge_mlp(edge_attr, We2, be2)
    h1 = _node1(aggp1, x, node2graph, W1a, b1a, W1b, b1b, gamma1, beta1, scale1)
    aggp2 = _sc_agg(h1, ea2, src, dst2)
    return _node2(aggp2, h1, x, W2a, b2a, W2b, b2b, scale2)


# speedup vs baseline: 4.1245x; 1.0873x over previous
import functools

import jax
import jax.numpy as jnp
from jax import lax
from jax.experimental import pallas as pl
from jax.experimental.pallas import tpu as pltpu
from jax.experimental.pallas import tpu_sc as plsc

_N = 10000
_E = 320000
_D = 128
_ED = 16
_G = 16

_NC = 2
_NS = 16
_EPT = _E // (_NC * _NS)
_K = 80
_NCH = _EPT // _K
_RPT = 624
_ZR = 16


def _sc_agg_body(x_hbm, ea_hbm, src_hbm, dst2_hbm, out_hbm,
                 srcall_v, dst2_v, data0, data1, zb_v, acc_sh,
                 sem_g0, sem_g1, sem_e0, sem_e1):
    cid = lax.axis_index("c")
    sid = lax.axis_index("s")
    ebase = (cid * _NS + sid) * _EPT
    tid = cid * _NS + sid

    pltpu.sync_copy(src_hbm.at[pl.ds(ebase, _EPT)], srcall_v)
    pltpu.sync_copy(dst2_hbm.at[tid], dst2_v)

    @pl.loop(0, _ZR)
    def _zfill(i):
        for j in range(_D // 16):
            zb_v[i, pl.ds(j * 16, 16)] = jnp.zeros((16,), jnp.float32)

    row0 = sid * _RPT
    nz = jnp.where(sid == _NS - 1, (_N - (_NS - 1) * _RPT) // _ZR, _RPT // _ZR)

    @pl.loop(0, nz)
    def _zcopy(k):
        pltpu.sync_copy(zb_v, acc_sh.at[pl.ds(row0 + k * _ZR, _ZR)])

    plsc.subcore_barrier()

    bufs = ((data0, sem_g0, sem_e0), (data1, sem_g1, sem_e1))

    def ea_slice(c):
        return ea_hbm.at[pl.ds(ebase + c * _K, _K)]

    def gx_src(c):
        return x_hbm.at[srcall_v.at[pl.ds(c * _K, _K)]]

    def relu_scatter(data, c):
        @pl.loop(0, _K, unroll=4)
        def _row(i):
            for j in range(_D // 16):
                s = pl.ds(j * 16, 16)
                data[i, s] = jnp.maximum(data[i, s], 0.0)

        pltpu.sync_copy(data, acc_sh.at[dst2_v.at[c]], add=True)

    pltpu.sync_copy(ea_slice(0), data0)
    pltpu.async_copy(gx_src(0), data0, sem_g0, add=True)
    pltpu.async_copy(ea_slice(1), data1, sem_e1)

    @pl.loop(0, (_NCH - 1) // 2)
    def _pair(p):
        for b in range(2):
            c = p * 2 + b
            data, sem_g, _ = bufs[b]
            dataN, sem_gN, sem_eN = bufs[1 - b]
            pltpu.make_async_copy(gx_src(c), data, sem_g).wait()
            pltpu.make_async_copy(ea_slice(c + 1), dataN, sem_eN).wait()
            pltpu.async_copy(gx_src(c + 1), dataN, sem_gN, add=True)
            relu_scatter(data, c)

            @pl.when(c + 2 < _NCH)
            def _():
                pltpu.async_copy(ea_slice(c + 2), data, bufs[b][2])

    c_last = _NCH - 1
    pltpu.make_async_copy(gx_src(c_last), data0, sem_g0).wait()
    relu_scatter(data0, c_last)

    plsc.subcore_barrier()

    @pl.loop(0, nz)
    def _wb(k):
        r = row0 + k * _ZR
        pltpu.sync_copy(acc_sh.at[pl.ds(r, _ZR)], zb_v)
        pltpu.sync_copy(zb_v, out_hbm.at[cid, pl.ds(r, _ZR)])


@functools.cache
def _make_sc_agg():
    return pl.kernel(
        _sc_agg_body,
        out_type=jax.ShapeDtypeStruct((_NC, _N, _D), jnp.float32),
        mesh=plsc.VectorSubcoreMesh(core_axis_name="c", subcore_axis_name="s",
                                    num_cores=_NC, num_subcores=_NS),
        scratch_types=[
            pltpu.VMEM((_EPT,), jnp.int32),
            pltpu.VMEM((_NCH, _K), jnp.int32),
            pltpu.VMEM((_K, _D), jnp.float32),
            pltpu.VMEM((_K, _D), jnp.float32),
            pltpu.VMEM((_ZR, _D), jnp.float32),
            pltpu.VMEM_SHARED((_N, _D), jnp.float32),
        ] + [pltpu.SemaphoreType.DMA] * 4,
    )


def _sc_agg(x, ea, src, dst2):
    return _make_sc_agg()(x, ea, src, dst2)


def _silu(x):
    return x * jax.nn.sigmoid(x)


_BE = 8000


def _edge_mlp_body(ea_ref, We_ref, be_ref, o_ref):
    a = ea_ref[...]
    h = jnp.dot(a, We_ref[...], preferred_element_type=jnp.float32) + be_ref[...]
    o_ref[...] = _silu(h)


def _edge_mlp(edge_attr, We, be):
    return pl.pallas_call(
        _edge_mlp_body,
        grid=(_E // _BE,),
        in_specs=[
            pl.BlockSpec((_BE, _ED), lambda i: (i, 0)),
            pl.BlockSpec((_ED, _D), lambda i: (0, 0)),
            pl.BlockSpec((1, _D), lambda i: (0, 0)),
        ],
        out_specs=pl.BlockSpec((_BE, _D), lambda i: (i, 0)),
        out_shape=jax.ShapeDtypeStruct((_E, _D), jnp.float32),
    )(edge_attr, We, be.reshape(1, _D))


def _node1_body(aggp_ref, x_ref, n2g_ref, Wa_ref, ba_ref, Wb_ref, bb_ref,
                gamma_ref, beta_ref, scale_ref, o_ref):
    s = scale_ref[0, 0]
    upd = aggp_ref[0] + aggp_ref[1] + s * x_ref[...]
    h = _silu(jnp.dot(upd, Wa_ref[...], preferred_element_type=jnp.float32, precision=lax.Precision.HIGHEST) + ba_ref[...])
    h = _silu(jnp.dot(h, Wb_ref[...], preferred_element_type=jnp.float32, precision=lax.Precision.HIGHEST) + bb_ref[...])
    gids = lax.broadcasted_iota(jnp.int32, (_N, _G), 1)
    onehot = (n2g_ref[...] == gids).astype(jnp.float32)
    ones = jnp.ones((_N, 1), jnp.float32)
    cnt = lax.dot_general(onehot, ones, (((0,), (0,)), ((), ())),
                          preferred_element_type=jnp.float32, precision=lax.Precision.HIGHEST)
    denom = jnp.maximum(cnt * float(_D), 1.0)
    sums = lax.dot_general(onehot, h, (((0,), (0,)), ((), ())),
                           preferred_element_type=jnp.float32, precision=lax.Precision.HIGHEST)
    mean = jnp.sum(sums, axis=1, keepdims=True) / denom
    mean_n = jnp.dot(onehot, mean, preferred_element_type=jnp.float32, precision=lax.Precision.HIGHEST)
    xc = h - mean_n
    sq = lax.dot_general(onehot, xc * xc, (((0,), (0,)), ((), ())),
                         preferred_element_type=jnp.float32, precision=lax.Precision.HIGHEST)
    var = jnp.sum(sq, axis=1, keepdims=True) / denom
    rs = lax.rsqrt(var + 1e-5)
    rs_n = jnp.dot(onehot, rs, preferred_element_type=jnp.float32, precision=lax.Precision.HIGHEST)
    out = (xc * rs_n) * gamma_ref[...] + beta_ref[...]
    o_ref[...] = jnp.maximum(out, 0.0)


def _node1(aggp, x, n2g, Wa, ba, Wb, bb, gamma, beta, scale):
    return pl.pallas_call(
        _node1_body,
        out_shape=jax.ShapeDtypeStruct((_N, _D), jnp.float32),
    )(aggp, x, n2g.reshape(_N, 1), Wa, ba.reshape(1, _D), Wb, bb.reshape(1, _D),
      gamma.reshape(1, _D), beta.reshape(1, _D), scale)


def _node2_body(aggp_ref, h1_ref, x_ref, Wa_ref, ba_ref, Wb_ref, bb_ref,
                scale_ref, o_ref):
    s = scale_ref[0, 0]
    upd = aggp_ref[0] + aggp_ref[1] + s * h1_ref[...]
    h = _silu(jnp.dot(upd, Wa_ref[...], preferred_element_type=jnp.float32, precision=lax.Precision.HIGHEST) + ba_ref[...])
    h = _silu(jnp.dot(h, Wb_ref[...], preferred_element_type=jnp.float32, precision=lax.Precision.HIGHEST) + bb_ref[...])
    o_ref[...] = jnp.maximum((h + x_ref[...]) * 0.5, 0.0)


def _node2(aggp, h1, x, Wa, ba, Wb, bb, scale):
    return pl.pallas_call(
        _node2_body,
        out_shape=jax.ShapeDtypeStruct((_N, _D), jnp.float32),
    )(aggp, h1, x, Wa, ba.reshape(1, _D), Wb, bb.reshape(1, _D), scale)


def kernel(x, edge_index, edge_attr, node2graph, We1, be1, eps1, W1a, b1a,
           W1b, b1b, gamma1, beta1, We2, be2, eps2, W2a, b2a, W2b, b2b):
    src = edge_index[0]
    dst2 = edge_index[1].reshape(_NC * _NS, _NCH, _K)
    scale1 = jnp.reshape(1.0 + eps1, (1, 1)).astype(jnp.float32)
    scale2 = jnp.reshape(1.0 + eps2, (1, 1)).astype(jnp.float32)

    ea1 = _edge_mlp(edge_attr, We1, be1)
    aggp1 = _sc_agg(x, ea1, src, dst2)
    ea2 = _edge_mlp(edge_attr, We2, be2)
    h1 = _node1(aggp1, x, node2graph, W1a, b1a, W1b, b1b, gamma1, beta1, scale1)
    aggp2 = _sc_agg(h1, ea2, src, dst2)
    return _node2(aggp2, h1, x, W2a, b2a, W2b, b2b, scale2)
